# Initial kernel scaffold; baseline (speedup 1.0000x reference)
import jax
import jax.numpy as jnp
from jax.experimental import pallas as pl


def kernel(X, edge_index, W_self, W_neigh, b_g, W_ih, W_hh, b_ih, b_hh, W_fc, b_fc):
    raise NotImplementedError("write your pallas kernel here")


# speedup vs baseline: 3.9423x; 3.9423x over previous
import jax
import jax.numpy as jnp
from jax import lax
from jax.experimental import pallas as pl
from jax.experimental.pallas import tpu as pltpu
from jax.experimental.pallas import tpu_sc as plsc

B, N, T_IN, T_OUT, F_IN, H = 2, 10000, 12, 3, 128, 64
E = 320000
BT = B * T_IN
CH = BT * H // 128
CPC = CH // 2
NC, NS = 2, 16
WPT = 160
NG, GW = 8, 20
EP = NS * WPT * 128
NP = 10240
ROWS_PT = NP // NS

def _mesh():
    return plsc.VectorSubcoreMesh(core_axis_name="c", subcore_axis_name="s",
                                  num_cores=NC, num_subcores=NS)


def _sc_deg_body(dst_hbm, z16_hbm, ones_hbm, deg_hbm, didx, dik, ob, dacc):
    c = lax.axis_index("c")
    s = lax.axis_index("s")
    rows0 = s * ROWS_PT
    pltpu.sync_copy(ones_hbm.at[pl.ds(0, 16)], ob)
    pltpu.sync_copy(z16_hbm, dacc.at[pl.ds(rows0, ROWS_PT)])
    plsc.subcore_barrier()
    for g in range(NG):
        pltpu.sync_copy(dst_hbm.at[s, g], didx)

        def dw(w, _):
            for k in range(8):
                dik[...] = didx[w, pl.ds(k * 16, 16)]
                pltpu.sync_copy(ob, dacc.at[dik], add=True)
            return 0
        lax.fori_loop(0, GW, dw, 0)
    plsc.subcore_barrier()
    pltpu.sync_copy(dacc.at[pl.ds(rows0, ROWS_PT)],
                    deg_hbm.at[c, pl.ds(rows0, ROWS_PT)])


def _sc_deg(dst_pad, z16, ones):
    return pl.kernel(
        _sc_deg_body,
        out_type=jax.ShapeDtypeStruct((NC, NP, 16), jnp.float32),
        mesh=_mesh(),
        scratch_types=[
            pltpu.VMEM((GW, 128), jnp.int32),
            pltpu.VMEM((16,), jnp.int32),
            pltpu.VMEM((16, 16), jnp.float32),
            pltpu.VMEM_SHARED((NP, 16), jnp.float32),
        ],
    )(dst_pad, z16, ones)


def _proj_body(x_ref, w2_ref, bg_ref, y3_ref, s_ref):
    x = x_ref[0]
    w2 = w2_ref[...]
    bg = bg_ref[...]
    r0 = jnp.dot(x[:, :F_IN], w2, preferred_element_type=jnp.float32)
    r1 = jnp.dot(x[:, F_IN:], w2, preferred_element_type=jnp.float32)
    y3_ref[0] = jnp.concatenate([r0[:, H:], r1[:, H:]], axis=1)
    s_ref[0] = jnp.concatenate([r0[:, :H] + bg, r1[:, :H] + bg], axis=1)


def _proj(Xf, w2, bg, bn=1000):
    grid = (CH, N // bn)
    return pl.pallas_call(
        _proj_body,
        grid=grid,
        in_specs=[
            pl.BlockSpec((1, bn, 2 * F_IN), lambda j, i: (j // CPC, i, j % CPC)),
            pl.BlockSpec((F_IN, 2 * H), lambda j, i: (0, 0)),
            pl.BlockSpec((1, H), lambda j, i: (0, 0)),
        ],
        out_specs=[
            pl.BlockSpec((1, bn, 128), lambda j, i: (j, i, 0)),
            pl.BlockSpec((1, bn, 2 * H), lambda j, i: (j // CPC, i, j % CPC)),
        ],
        out_shape=[
            jax.ShapeDtypeStruct((CH, N, 128), jnp.float32),
            jax.ShapeDtypeStruct((B, N, T_IN * H), jnp.float32),
        ],
    )(Xf, w2, bg)


def _gru_body(s_ref, agg_ref, deg_ref, wih_ref, whh_ref, bih_ref, bhh_ref,
              wfc_ref, bfc_ref, gcn_ref, rnn_ref):
    deg = deg_ref[0, :, 0:1]
    dinv = 1.0 / jnp.maximum(deg, 1.0)
    xs = []
    for t in range(T_IN):
        cols = []
        for b in range(B):
            j = b * CPC + t // 2
            half = t % 2
            a = agg_ref[j, :, half * H:(half + 1) * H]
            g = jnp.maximum(s_ref[b, :, t * H:(t + 1) * H] + a * dinv, 0.0)
            gcn_ref[b, :, t, :] = g
            cols.append(g)
        xs.append(jnp.concatenate(cols, axis=0))

    wih = wih_ref[...]
    whh = whh_ref[...]
    bih = bih_ref[...]
    bhh = bhh_ref[...]
    h = jnp.zeros_like(xs[0])
    for t in range(T_IN):
        gi = jnp.dot(xs[t], wih, preferred_element_type=jnp.float32) + bih
        gh = jnp.dot(h, whh, preferred_element_type=jnp.float32) + bhh
        r = jax.nn.sigmoid(gi[:, :H] + gh[:, :H])
        z = jax.nn.sigmoid(gi[:, H:2 * H] + gh[:, H:2 * H])
        n = jnp.tanh(gi[:, 2 * H:] + r * gh[:, 2 * H:])
        h = (1.0 - z) * n + z * h
    out = jnp.dot(h, wfc_ref[...], preferred_element_type=jnp.float32) \
        + bfc_ref[...]
    bn = deg.shape[0]
    rnn_ref[0] = out[:bn, :T_OUT]
    rnn_ref[1] = out[bn:, :T_OUT]


def _gru(S, aggp, degp, wih, whh, bih, bhh, wfcp, bfcp, bn=400):
    grid = (N // bn,)
    return pl.pallas_call(
        _gru_body,
        grid=grid,
        in_specs=[
            pl.BlockSpec((B, bn, T_IN * H), lambda i: (0, i, 0)),
            pl.BlockSpec((CH, bn, 128), lambda i: (0, i, 0)),
            pl.BlockSpec((1, bn, 16), lambda i: (0, i, 0)),
            pl.BlockSpec((H, 3 * H), lambda i: (0, 0)),
            pl.BlockSpec((H, 3 * H), lambda i: (0, 0)),
            pl.BlockSpec((1, 3 * H), lambda i: (0, 0)),
            pl.BlockSpec((1, 3 * H), lambda i: (0, 0)),
            pl.BlockSpec((H, 128), lambda i: (0, 0)),
            pl.BlockSpec((1, 128), lambda i: (0, 0)),
        ],
        out_specs=[
            pl.BlockSpec((B, bn, T_IN, H), lambda i: (0, i, 0, 0)),
            pl.BlockSpec((B, bn, T_OUT), lambda i: (0, i, 0)),
        ],
        out_shape=[
            jax.ShapeDtypeStruct((B, N, T_IN, H), jnp.float32),
            jax.ShapeDtypeStruct((B, N, T_OUT), jnp.float32),
        ],
    )(S, aggp, degp, wih, whh, bih, bhh, wfcp, bfcp)


def kernel(X, edge_index, W_self, W_neigh, b_g, W_ih, W_hh, b_ih, b_hh,
           W_fc, b_fc):
    src = edge_index[0]
    dst = edge_index[1]
    pad = EP - E
    lane = jnp.arange(pad, dtype=jnp.int32) % 128
    dst_pad = jnp.concatenate([dst, N + lane]).reshape(NS, NG, GW, 128)

    w2 = jnp.concatenate([W_self, W_neigh], axis=1)
    bg = b_g.reshape(1, H)
    Y3, S = _proj(X.reshape(B, N, T_IN * F_IN), w2, bg)

    deg1 = jax.ops.segment_sum(jnp.ones((E,), jnp.float32), dst,
                               num_segments=NP)
    degp = jnp.broadcast_to(deg1[None, :, None], (1, NP, 16))

    Yf = Y3.reshape(CH * N, 128)
    aggp = jnp.stack([
        jax.ops.segment_sum(Yf[ch * N + src], dst, num_segments=NP)
        for ch in range(CH)])

    wfcp = jnp.zeros((H, 128), jnp.float32).at[:, :T_OUT].set(W_fc)
    bfcp = jnp.zeros((1, 128), jnp.float32).at[0, :T_OUT].set(b_fc)
    gcn_out, rnn_out = _gru(S, aggp, degp, W_ih, W_hh,
                            b_ih.reshape(1, 3 * H), b_hh.reshape(1, 3 * H),
                            wfcp, bfcp)
    return rnn_out, gcn_out
